# initial kernel scaffold (unmeasured)
import jax
import jax.numpy as jnp
from jax import lax
from jax.experimental import pallas as pl
from jax.experimental.pallas import tpu as pltpu

N_DEV = 4
N_BLK = 1024
COMM_DTYPE = jnp.bfloat16


def kernel(x, w_mat, scale_x, scale_w):
    m_total, _k = x.shape
    _k2, n = w_mat.shape
    m_per = m_total // N_DEV
    nb = n // N_BLK
    n_hops = nb * (N_DEV - 1)

    scale = (scale_x * scale_w).reshape(1, 1)

    def body(x_ref, w_ref, scale_ref, out_ref,
             comm_ref, send_sems, recv_sems, ready_sem):
        d = lax.axis_index("i")
        right = lax.rem(d + 1, N_DEV)
        left = lax.rem(d + N_DEV - 1, N_DEV)

        barrier_sem = pltpu.get_barrier_semaphore()
        for nbr in (left, right):
            pl.semaphore_signal(barrier_sem, inc=1, device_id=(nbr,),
                                device_id_type=pl.DeviceIdType.MESH)
        pl.semaphore_wait(barrier_sem, 2)

        pl.semaphore_signal(ready_sem, inc=1, device_id=(left,),
                            device_id_type=pl.DeviceIdType.MESH)

        def partial(chunk_idx, blk):
            xc = x_ref[pl.ds(chunk_idx * m_per, m_per), :].astype(jnp.bfloat16)
            wc = w_ref[:, pl.ds(blk * N_BLK, N_BLK)].astype(jnp.bfloat16)
            return lax.dot_general(xc, wc, (((1,), (0,)), ((), ())),
                                   preferred_element_type=jnp.float32)

        H = 0
        for blk in range(nb):
            c0 = lax.rem(d + N_DEV - 1, N_DEV)
            comm_ref[H % 2] = partial(c0, blk).astype(COMM_DTYPE)
            for s in range(N_DEV - 1):
                send_slot = H % 2
                recv_slot = (H + 1) % 2
                pl.semaphore_wait(ready_sem, 1)
                rdma = pltpu.make_async_remote_copy(
                    src_ref=comm_ref.at[send_slot],
                    dst_ref=comm_ref.at[recv_slot],
                    send_sem=send_sems.at[send_slot],
                    recv_sem=recv_sems.at[recv_slot],
                    device_id=(right,),
                    device_id_type=pl.DeviceIdType.MESH,
                )
                rdma.start()
                rdma.wait()
                c = lax.rem(d + 2 * N_DEV - 2 - s, N_DEV)
                acc = comm_ref[recv_slot].astype(jnp.float32) + partial(c, blk)
                if s < N_DEV - 2:
                    comm_ref[recv_slot] = acc.astype(COMM_DTYPE)
                else:
                    out_ref[:, pl.ds(blk * N_BLK, N_BLK)] = jnp.maximum(
                        acc * scale_ref[0, 0], 0.0)
                H += 1
                if H < n_hops:
                    pl.semaphore_signal(ready_sem, inc=1, device_id=(left,),
                                        device_id_type=pl.DeviceIdType.MESH)

    return pl.pallas_call(
        body,
        out_shape=jax.ShapeDtypeStruct((m_per, n), jnp.float32),
        in_specs=[
            pl.BlockSpec(memory_space=pltpu.VMEM),
            pl.BlockSpec(memory_space=pltpu.VMEM),
            pl.BlockSpec(memory_space=pltpu.SMEM),
        ],
        out_specs=pl.BlockSpec(memory_space=pltpu.VMEM),
        scratch_shapes=[
            pltpu.VMEM((2, m_per, N_BLK), COMM_DTYPE),
            pltpu.SemaphoreType.DMA((2,)),
            pltpu.SemaphoreType.DMA((2,)),
            pltpu.SemaphoreType.REGULAR,
        ],
        compiler_params=pltpu.CompilerParams(collective_id=0),
    )(x, w_mat, scale)


# baseline (device time: 720588 ns/iter reference)
import jax
import jax.numpy as jnp
from jax import lax
from jax.experimental import pallas as pl
from jax.experimental.pallas import tpu as pltpu

N_DEV = 4
N_BLK = 1024
COMM_DTYPE = jnp.bfloat16


def kernel(x, w_mat, scale_x, scale_w):
    m_total, _k = x.shape
    _k2, n = w_mat.shape
    m_per = m_total // N_DEV
    nb = n // N_BLK
    n_hops = nb * (N_DEV - 1)

    x = x.astype(jnp.float8_e4m3fn)
    w_mat = w_mat.astype(jnp.float8_e5m2)
    scale = (scale_x * scale_w).reshape(1, 1)

    def body(x_ref, w_ref, scale_ref, out_ref,
             comm_ref, stage_ref, send_sems, recv_sems, out_sems, ready_sem):
        d = lax.axis_index("i")
        right = lax.rem(d + 1, N_DEV)
        left = lax.rem(d + N_DEV - 1, N_DEV)

        barrier_sem = pltpu.get_barrier_semaphore()
        for nbr in (left, right):
            pl.semaphore_signal(barrier_sem, inc=1, device_id=(nbr,),
                                device_id_type=pl.DeviceIdType.MESH)
        pl.semaphore_wait(barrier_sem, 2)

        pl.semaphore_signal(ready_sem, inc=1, device_id=(left,),
                            device_id_type=pl.DeviceIdType.MESH)

        def partial(chunk_idx, blk):
            xc = x_ref[pl.ds(chunk_idx * m_per, m_per), :].astype(jnp.bfloat16)
            wc = w_ref[:, pl.ds(blk * N_BLK, N_BLK)].astype(jnp.bfloat16)
            return lax.dot_general(xc, wc, (((1,), (0,)), ((), ())),
                                   preferred_element_type=jnp.float32)

        out_copies = {}
        H = 0
        for blk in range(nb):
            c0 = lax.rem(d + N_DEV - 1, N_DEV)
            comm_ref[H % 2] = partial(c0, blk).astype(COMM_DTYPE)
            for s in range(N_DEV - 1):
                send_slot = H % 2
                recv_slot = (H + 1) % 2
                pl.semaphore_wait(ready_sem, 1)
                rdma = pltpu.make_async_remote_copy(
                    src_ref=comm_ref.at[send_slot],
                    dst_ref=comm_ref.at[recv_slot],
                    send_sem=send_sems.at[send_slot],
                    recv_sem=recv_sems.at[recv_slot],
                    device_id=(right,),
                    device_id_type=pl.DeviceIdType.MESH,
                )
                rdma.start()
                rdma.wait()
                c = lax.rem(d + 2 * N_DEV - 2 - s, N_DEV)
                acc = comm_ref[recv_slot].astype(jnp.float32) + partial(c, blk)
                if s < N_DEV - 2:
                    comm_ref[recv_slot] = acc.astype(COMM_DTYPE)
                else:
                    oslot = blk % 2
                    if blk >= 2:
                        out_copies[blk - 2].wait()
                    stage_ref[oslot] = jnp.maximum(acc * scale_ref[0, 0], 0.0)
                    cp = pltpu.make_async_copy(
                        stage_ref.at[oslot],
                        out_ref.at[:, pl.ds(blk * N_BLK, N_BLK)],
                        out_sems.at[oslot],
                    )
                    cp.start()
                    out_copies[blk] = cp
                H += 1
                if H < n_hops:
                    pl.semaphore_signal(ready_sem, inc=1, device_id=(left,),
                                        device_id_type=pl.DeviceIdType.MESH)
        for blk in (nb - 2, nb - 1):
            out_copies[blk].wait()

    return pl.pallas_call(
        body,
        out_shape=jax.ShapeDtypeStruct((m_per, n), jnp.float32),
        in_specs=[
            pl.BlockSpec(memory_space=pltpu.VMEM),
            pl.BlockSpec(memory_space=pltpu.VMEM),
            pl.BlockSpec(memory_space=pltpu.SMEM),
        ],
        out_specs=pl.BlockSpec(memory_space=pl.ANY),
        scratch_shapes=[
            pltpu.VMEM((2, m_per, N_BLK), COMM_DTYPE),
            pltpu.VMEM((2, m_per, N_BLK), jnp.float32),
            pltpu.SemaphoreType.DMA((2,)),
            pltpu.SemaphoreType.DMA((2,)),
            pltpu.SemaphoreType.DMA((2,)),
            pltpu.SemaphoreType.REGULAR,
        ],
        compiler_params=pltpu.CompilerParams(collective_id=0),
    )(x, w_mat, scale)


# device time: 380398 ns/iter; 1.8943x vs baseline; 1.8943x over previous
import jax
import jax.numpy as jnp
from jax import lax
from jax.experimental import pallas as pl
from jax.experimental.pallas import tpu as pltpu

N_DEV = 4
N_SUB = 1024
COMM_DTYPE = jnp.bfloat16


def kernel(x, w_mat, scale_x, scale_w):
    m_total, _k = x.shape
    _k2, n = w_mat.shape
    m_per = m_total // N_DEV
    half = n // 2
    nb = half // N_SUB
    n_hops = nb * (N_DEV - 1)

    x = x.astype(jnp.float8_e4m3fn)
    w_mat = w_mat.astype(jnp.float8_e5m2)
    scale = (scale_x * scale_w).reshape(1, 1)

    def body(x_ref, w_ref, scale_ref, out_ref,
             comm_r, comm_l, stage_r, stage_l,
             send_r, recv_r, send_l, recv_l, outs_r, outs_l,
             ready_r, ready_l):
        d = lax.axis_index("i")
        right = lax.rem(d + 1, N_DEV)
        left = lax.rem(d + N_DEV - 1, N_DEV)

        barrier_sem = pltpu.get_barrier_semaphore()
        for nbr in (left, right):
            pl.semaphore_signal(barrier_sem, inc=1, device_id=(nbr,),
                                device_id_type=pl.DeviceIdType.MESH)
        pl.semaphore_wait(barrier_sem, 2)

        def partial(chunk_idx, col_off):
            xc = x_ref[pl.ds(chunk_idx * m_per, m_per), :].astype(jnp.bfloat16)
            wc = w_ref[:, pl.ds(col_off, N_SUB)].astype(jnp.bfloat16)
            return lax.dot_general(xc, wc, (((1,), (0,)), ((), ())),
                                   preferred_element_type=jnp.float32)

        dirs = [
            dict(comm=comm_r, send=send_r, recv=recv_r, outs=outs_r,
                 ready=ready_r, tgt=right, upstream=left, base=0,
                 seed_c=lax.rem(d + N_DEV - 1, N_DEV),
                 in_c=lambda s: lax.rem(d + 2 * N_DEV - 2 - s, N_DEV)),
            dict(comm=comm_l, send=send_l, recv=recv_l, outs=outs_l,
                 ready=ready_l, tgt=left, upstream=right, base=half,
                 seed_c=lax.rem(d + 1, N_DEV),
                 in_c=lambda s: lax.rem(d + 2 + s, N_DEV)),
        ]
        for dr in dirs:
            pl.semaphore_signal(ready_sem := dr["ready"], inc=1,
                                device_id=(dr["upstream"],),
                                device_id_type=pl.DeviceIdType.MESH)
            dr["copies"] = {}

        H = 0
        for blk in range(nb):
            for dr in dirs:
                dr["comm"][H % 2] = partial(
                    dr["seed_c"], dr["base"] + blk * N_SUB).astype(COMM_DTYPE)
            for s in range(N_DEV - 1):
                send_slot = H % 2
                recv_slot = (H + 1) % 2
                rdmas = []
                for dr in dirs:
                    pl.semaphore_wait(dr["ready"], 1)
                    rdma = pltpu.make_async_remote_copy(
                        src_ref=dr["comm"].at[send_slot],
                        dst_ref=dr["comm"].at[recv_slot],
                        send_sem=dr["send"].at[send_slot],
                        recv_sem=dr["recv"].at[recv_slot],
                        device_id=(dr["tgt"],),
                        device_id_type=pl.DeviceIdType.MESH,
                    )
                    rdma.start()
                    rdmas.append(rdma)
                parts = [partial(dr["in_c"](s), dr["base"] + blk * N_SUB)
                         for dr in dirs]
                for dr, rdma, p in zip(dirs, rdmas, parts):
                    rdma.wait_recv()
                    acc = dr["comm"][recv_slot].astype(jnp.float32) + p
                    if s < N_DEV - 2:
                        dr["comm"][recv_slot] = acc.astype(COMM_DTYPE)
                    else:
                        oslot = blk % 2
                        if blk >= 2:
                            dr["copies"][blk - 2].wait()
                        stage = stage_r if dr["base"] == 0 else stage_l
                        stage[oslot] = jnp.maximum(acc * scale_ref[0, 0], 0.0)
                        cp = pltpu.make_async_copy(
                            stage.at[oslot],
                            out_ref.at[:, pl.ds(dr["base"] + blk * N_SUB,
                                                N_SUB)],
                            dr["outs"].at[oslot],
                        )
                        cp.start()
                        dr["copies"][blk] = cp
                    rdma.wait_send()
                    if H + 1 < n_hops:
                        pl.semaphore_signal(dr["ready"], inc=1,
                                            device_id=(dr["upstream"],),
                                            device_id_type=pl.DeviceIdType.MESH)
                H += 1
        for dr in dirs:
            for blk in range(max(0, nb - 2), nb):
                dr["copies"][blk].wait()

    return pl.pallas_call(
        body,
        out_shape=jax.ShapeDtypeStruct((m_per, n), jnp.float32),
        in_specs=[
            pl.BlockSpec(memory_space=pltpu.VMEM),
            pl.BlockSpec(memory_space=pltpu.VMEM),
            pl.BlockSpec(memory_space=pltpu.SMEM),
        ],
        out_specs=pl.BlockSpec(memory_space=pl.ANY),
        scratch_shapes=[
            pltpu.VMEM((2, m_per, N_SUB), COMM_DTYPE),
            pltpu.VMEM((2, m_per, N_SUB), COMM_DTYPE),
            pltpu.VMEM((2, m_per, N_SUB), jnp.float32),
            pltpu.VMEM((2, m_per, N_SUB), jnp.float32),
            pltpu.SemaphoreType.DMA((2,)),
            pltpu.SemaphoreType.DMA((2,)),
            pltpu.SemaphoreType.DMA((2,)),
            pltpu.SemaphoreType.DMA((2,)),
            pltpu.SemaphoreType.DMA((2,)),
            pltpu.SemaphoreType.DMA((2,)),
            pltpu.SemaphoreType.REGULAR,
            pltpu.SemaphoreType.REGULAR,
        ],
        compiler_params=pltpu.CompilerParams(
            collective_id=0, vmem_limit_bytes=50 * 1024 * 1024),
    )(x, w_mat, scale)


# device time: 341409 ns/iter; 2.1106x vs baseline; 1.1142x over previous
import jax
import jax.numpy as jnp
from jax import lax
from jax.experimental import pallas as pl
from jax.experimental.pallas import tpu as pltpu

N_DEV = 4
N_SUB = 1024
N_CHAIN = 2
COMM_DTYPE = jnp.bfloat16


def kernel(x, w_mat, scale_x, scale_w):
    m_total, _k = x.shape
    _k2, n = w_mat.shape
    m_per = m_total // N_DEV
    half = n // 2
    nb = half // N_SUB
    hops_per_ring = (nb // N_CHAIN) * (N_DEV - 1)

    x = x.astype(jnp.float8_e4m3fn)
    w_mat = w_mat.astype(jnp.float8_e5m2)
    scale = (scale_x * scale_w).reshape(1, 1)

    def body(x_ref, w_ref, scale_ref, out_ref,
             comm0, comm1, comm2, comm3, stage_r, stage_l,
             send_sems, recv_sems, out_sems,
             ready0, ready1, ready2, ready3):
        d = lax.axis_index("i")
        right = lax.rem(d + 1, N_DEV)
        left = lax.rem(d + N_DEV - 1, N_DEV)

        barrier_sem = pltpu.get_barrier_semaphore()
        for nbr in (left, right):
            pl.semaphore_signal(barrier_sem, inc=1, device_id=(nbr,),
                                device_id_type=pl.DeviceIdType.MESH)
        pl.semaphore_wait(barrier_sem, 2)

        def partial(chunk_idx, col_off):
            xc = x_ref[pl.ds(chunk_idx * m_per, m_per), :].astype(jnp.bfloat16)
            wc = w_ref[:, pl.ds(col_off, N_SUB)].astype(jnp.bfloat16)
            return lax.dot_general(xc, wc, (((1,), (0,)), ((), ())),
                                   preferred_element_type=jnp.float32)

        dir_specs = [
            dict(tgt=right, upstream=left, base=0, stage=stage_r,
                 seed_c=lax.rem(d + N_DEV - 1, N_DEV),
                 in_c=lambda s: lax.rem(d + 2 * N_DEV - 2 - s, N_DEV)),
            dict(tgt=left, upstream=right, base=half, stage=stage_l,
                 seed_c=lax.rem(d + 1, N_DEV),
                 in_c=lambda s: lax.rem(d + 2 + s, N_DEV)),
        ]
        comms = [comm0, comm1, comm2, comm3]
        readys = [ready0, ready1, ready2, ready3]
        dir_copies = [[], []]

        rings = []
        for di, ds_ in enumerate(dir_specs):
            for ch in range(N_CHAIN):
                r = di * N_CHAIN + ch
                blocks = list(range(ch, nb, N_CHAIN))
                rings.append(dict(
                    ds_, idx=r, dir=di, comm=comms[r], ready=readys[r],
                    tokens=[(b, s) for b in blocks for s in range(N_DEV - 1)],
                    H=0, rdma=None, p=None,
                ))
                pl.semaphore_signal(readys[r], inc=1,
                                    device_id=(ds_["upstream"],),
                                    device_id_type=pl.DeviceIdType.MESH)

        def col(ring, blk):
            return ring["base"] + blk * N_SUB

        def seed(ring, blk):
            ring["comm"][ring["H"] % 2] = partial(
                ring["seed_c"], col(ring, blk)).astype(COMM_DTYPE)

        def start(ring):
            H = ring["H"]
            pl.semaphore_wait(ring["ready"], 1)
            rdma = pltpu.make_async_remote_copy(
                src_ref=ring["comm"].at[H % 2],
                dst_ref=ring["comm"].at[(H + 1) % 2],
                send_sem=send_sems.at[ring["idx"], H % 2],
                recv_sem=recv_sems.at[ring["idx"], (H + 1) % 2],
                device_id=(ring["tgt"],),
                device_id_type=pl.DeviceIdType.MESH,
            )
            rdma.start()
            ring["rdma"] = rdma

        def stash(ring, blk, s):
            ring["p"] = partial(ring["in_c"](s), col(ring, blk)).astype(
                COMM_DTYPE)

        def finish(ring, blk, s):
            H = ring["H"]
            rdma = ring["rdma"]
            rdma.wait_recv()
            acc = (ring["comm"][(H + 1) % 2].astype(jnp.float32)
                   + ring["p"].astype(jnp.float32))
            if s < N_DEV - 2:
                ring["comm"][(H + 1) % 2] = acc.astype(COMM_DTYPE)
            else:
                di = ring["dir"]
                if dir_copies[di]:
                    dir_copies[di][-1].wait()
                ring["stage"][...] = jnp.maximum(acc * scale_ref[0, 0], 0.0)
                cp = pltpu.make_async_copy(
                    ring["stage"],
                    out_ref.at[:, pl.ds(col(ring, blk), N_SUB)],
                    out_sems.at[di],
                )
                cp.start()
                dir_copies[di].append(cp)
            rdma.wait_send()
            ring["H"] = H + 1
            if ring["H"] < hops_per_ring:
                pl.semaphore_signal(ring["ready"], inc=1,
                                    device_id=(ring["upstream"],),
                                    device_id_type=pl.DeviceIdType.MESH)

        for ring in rings:
            blk, s = ring["tokens"][0]
            seed(ring, blk)
            start(ring)
            stash(ring, blk, s)
        for t in range(hops_per_ring):
            for ring in rings:
                blk, s = ring["tokens"][t]
                finish(ring, blk, s)
                if t + 1 < hops_per_ring:
                    nblk, ns = ring["tokens"][t + 1]
                    if ns == 0:
                        seed(ring, nblk)
                    start(ring)
                    stash(ring, nblk, ns)
        for copies in dir_copies:
            copies[-1].wait()

    return pl.pallas_call(
        body,
        out_shape=jax.ShapeDtypeStruct((m_per, n), jnp.float32),
        in_specs=[
            pl.BlockSpec(memory_space=pltpu.VMEM),
            pl.BlockSpec(memory_space=pltpu.VMEM),
            pl.BlockSpec(memory_space=pltpu.SMEM),
        ],
        out_specs=pl.BlockSpec(memory_space=pl.ANY),
        scratch_shapes=[
            pltpu.VMEM((2, m_per, N_SUB), COMM_DTYPE),
            pltpu.VMEM((2, m_per, N_SUB), COMM_DTYPE),
            pltpu.VMEM((2, m_per, N_SUB), COMM_DTYPE),
            pltpu.VMEM((2, m_per, N_SUB), COMM_DTYPE),
            pltpu.VMEM((m_per, N_SUB), jnp.float32),
            pltpu.VMEM((m_per, N_SUB), jnp.float32),
            pltpu.SemaphoreType.DMA((4, 2)),
            pltpu.SemaphoreType.DMA((4, 2)),
            pltpu.SemaphoreType.DMA((2,)),
            pltpu.SemaphoreType.REGULAR,
            pltpu.SemaphoreType.REGULAR,
            pltpu.SemaphoreType.REGULAR,
            pltpu.SemaphoreType.REGULAR,
        ],
        compiler_params=pltpu.CompilerParams(
            collective_id=0, vmem_limit_bytes=50 * 1024 * 1024),
    )(x, w_mat, scale)


# device time: 336806 ns/iter; 2.1395x vs baseline; 1.0137x over previous
import jax
import jax.numpy as jnp
from jax import lax
from jax.experimental import pallas as pl
from jax.experimental.pallas import tpu as pltpu

N_DEV = 4
N_SUB = 1024
N_CHAIN = 2
COMM_DTYPE = jnp.bfloat16


def kernel(x, w_mat, scale_x, scale_w):
    m_total, _k = x.shape
    _k2, n = w_mat.shape
    m_per = m_total // N_DEV
    half = n // 2
    nb = half // N_SUB
    hops_per_ring = (nb // N_CHAIN) * (N_DEV - 1)

    x = x.astype(jnp.float8_e4m3fn)
    w_mat = w_mat.astype(jnp.float8_e5m2)
    scale = (scale_x * scale_w).reshape(1, 1)

    def body(x_ref, w_ref, scale_ref, out_ref,
             comm0, comm1, comm2, comm3, stage,
             send_sems, recv_sems, out_sem,
             ready0, ready1, ready2, ready3):
        d = lax.axis_index("i")
        right = lax.rem(d + 1, N_DEV)
        left = lax.rem(d + N_DEV - 1, N_DEV)

        barrier_sem = pltpu.get_barrier_semaphore()
        for nbr in (left, right):
            pl.semaphore_signal(barrier_sem, inc=1, device_id=(nbr,),
                                device_id_type=pl.DeviceIdType.MESH)
        pl.semaphore_wait(barrier_sem, 2)

        def partial(chunk_idx, col_off):
            xc = x_ref[pl.ds(chunk_idx * m_per, m_per), :]
            wc = w_ref[:, pl.ds(col_off, N_SUB)]
            return lax.dot_general(xc, wc, (((1,), (0,)), ((), ())),
                                   preferred_element_type=jnp.float32)

        dir_specs = [
            dict(tgt=right, upstream=left, base=0,
                 seed_c=lax.rem(d + N_DEV - 1, N_DEV),
                 in_c=lambda s: lax.rem(d + 2 * N_DEV - 2 - s, N_DEV)),
            dict(tgt=left, upstream=right, base=half,
                 seed_c=lax.rem(d + 1, N_DEV),
                 in_c=lambda s: lax.rem(d + 2 + s, N_DEV)),
        ]
        comms = [comm0, comm1, comm2, comm3]
        readys = [ready0, ready1, ready2, ready3]
        out_copies = []

        rings = []
        for di, ds_ in enumerate(dir_specs):
            for ch in range(N_CHAIN):
                r = di * N_CHAIN + ch
                blocks = list(range(ch, nb, N_CHAIN))
                rings.append(dict(
                    ds_, idx=r, dir=di, comm=comms[r], ready=readys[r],
                    tokens=[(b, s) for b in blocks for s in range(N_DEV - 1)],
                    H=0, rdma=None, p=None,
                ))
                pl.semaphore_signal(readys[r], inc=1,
                                    device_id=(ds_["upstream"],),
                                    device_id_type=pl.DeviceIdType.MESH)

        def col(ring, blk):
            return ring["base"] + blk * N_SUB

        def seed(ring, blk):
            ring["comm"][ring["H"] % 2] = partial(
                ring["seed_c"], col(ring, blk)).astype(COMM_DTYPE)

        def start(ring):
            H = ring["H"]
            pl.semaphore_wait(ring["ready"], 1)
            rdma = pltpu.make_async_remote_copy(
                src_ref=ring["comm"].at[H % 2],
                dst_ref=ring["comm"].at[(H + 1) % 2],
                send_sem=send_sems.at[ring["idx"], H % 2],
                recv_sem=recv_sems.at[ring["idx"], (H + 1) % 2],
                device_id=(ring["tgt"],),
                device_id_type=pl.DeviceIdType.MESH,
            )
            rdma.start()
            ring["rdma"] = rdma

        def stash(ring, blk, s):
            ring["p"] = partial(ring["in_c"](s), col(ring, blk)).astype(
                COMM_DTYPE)

        def finish(ring, blk, s):
            H = ring["H"]
            rdma = ring["rdma"]
            rdma.wait_recv()
            if s < N_DEV - 2:
                ring["comm"][(H + 1) % 2] = (
                    ring["comm"][(H + 1) % 2] + ring["p"])
            else:
                acc = (ring["comm"][(H + 1) % 2].astype(jnp.float32)
                       + ring["p"].astype(jnp.float32))
                if out_copies:
                    out_copies[-1].wait()
                stage[...] = jnp.maximum(acc * scale_ref[0, 0], 0.0)
                cp = pltpu.make_async_copy(
                    stage,
                    out_ref.at[:, pl.ds(col(ring, blk), N_SUB)],
                    out_sem,
                )
                cp.start()
                out_copies.append(cp)
            rdma.wait_send()
            ring["H"] = H + 1
            if ring["H"] < hops_per_ring:
                pl.semaphore_signal(ring["ready"], inc=1,
                                    device_id=(ring["upstream"],),
                                    device_id_type=pl.DeviceIdType.MESH)

        for ring in rings:
            blk, s = ring["tokens"][0]
            seed(ring, blk)
            start(ring)
            stash(ring, blk, s)
        for t in range(hops_per_ring):
            for ring in rings:
                blk, s = ring["tokens"][t]
                finish(ring, blk, s)
                if t + 1 < hops_per_ring:
                    nblk, ns = ring["tokens"][t + 1]
                    if ns == 0:
                        seed(ring, nblk)
                    start(ring)
                    stash(ring, nblk, ns)
        out_copies[-1].wait()

    return pl.pallas_call(
        body,
        out_shape=jax.ShapeDtypeStruct((m_per, n), jnp.float32),
        in_specs=[
            pl.BlockSpec(memory_space=pltpu.VMEM),
            pl.BlockSpec(memory_space=pltpu.VMEM),
            pl.BlockSpec(memory_space=pltpu.SMEM),
        ],
        out_specs=pl.BlockSpec(memory_space=pl.ANY),
        scratch_shapes=[
            pltpu.VMEM((2, m_per, N_SUB), COMM_DTYPE),
            pltpu.VMEM((2, m_per, N_SUB), COMM_DTYPE),
            pltpu.VMEM((2, m_per, N_SUB), COMM_DTYPE),
            pltpu.VMEM((2, m_per, N_SUB), COMM_DTYPE),
            pltpu.VMEM((m_per, N_SUB), jnp.float32),
            pltpu.SemaphoreType.DMA((4, 2)),
            pltpu.SemaphoreType.DMA((4, 2)),
            pltpu.SemaphoreType.DMA,
            pltpu.SemaphoreType.REGULAR,
            pltpu.SemaphoreType.REGULAR,
            pltpu.SemaphoreType.REGULAR,
            pltpu.SemaphoreType.REGULAR,
        ],
        compiler_params=pltpu.CompilerParams(
            collective_id=0, vmem_limit_bytes=50 * 1024 * 1024),
    )(x, w_mat, scale)
